# Initial kernel scaffold; baseline (speedup 1.0000x reference)
import jax
import jax.numpy as jnp
from jax.experimental import pallas as pl


def kernel(x, edge_index, edge_weight, W):
    raise NotImplementedError("write your pallas kernel here")


# speedup vs baseline: 4.1603x; 4.1603x over previous
import functools

import jax
import jax.numpy as jnp
from jax import lax
from jax.experimental import pallas as pl
from jax.experimental.pallas import tpu as pltpu
from jax.experimental.pallas import tpu_sc as plsc

N_NODES = 10000
N_EDGES = 320000
D = 128

NC = 2
NS = 16
NW = NC * NS
E_PER_TILE = N_EDGES // NW
CHUNK = 80
N_CHUNKS = E_PER_TILE // CHUNK
N_PAD = 10240
ROWS_PER_TILE = N_PAD // NS


def _sc_scatter(x, row, col, ew, zeros):
    mesh = plsc.VectorSubcoreMesh(core_axis_name="c", subcore_axis_name="s")

    @functools.partial(
        pl.kernel,
        mesh=mesh,
        out_type=jax.ShapeDtypeStruct((NC, N_PAD, D), jnp.float32),
        scratch_types=[
            pltpu.VMEM_SHARED((N_PAD, D), jnp.float32),
            pltpu.VMEM((CHUNK,), jnp.int32),
            pltpu.VMEM((CHUNK,), jnp.int32),
            pltpu.VMEM((CHUNK,), jnp.float32),
            pltpu.VMEM((CHUNK, D), jnp.float32),
            pltpu.SemaphoreType.DMA,
        ],
    )
    def k(x_hbm, row_hbm, col_hbm, ew_hbm, z_hbm, out_hbm,
          acc, col_v, row_v, ew_v, rows_v, sem):
        cid = lax.axis_index("c")
        sid = lax.axis_index("s")
        wid = cid * NS + sid

        rbase = sid * ROWS_PER_TILE
        pltpu.sync_copy(z_hbm.at[pl.ds(rbase, ROWS_PER_TILE)],
                        acc.at[pl.ds(rbase, ROWS_PER_TILE)])
        plsc.subcore_barrier()

        ebase = wid * E_PER_TILE

        def chunk_body(i, carry):
            off = pl.multiple_of(ebase + i * CHUNK, 8)
            pltpu.sync_copy(col_hbm.at[pl.ds(off, CHUNK)], col_v)
            pltpu.sync_copy(row_hbm.at[pl.ds(off, CHUNK)], row_v)
            pltpu.sync_copy(ew_hbm.at[pl.ds(off, CHUNK)], ew_v)
            pltpu.async_copy(x_hbm.at[col_v], rows_v, sem).wait()

            def scale_body(g, c):
                w16 = ew_v[pl.ds(g * 16, 16)]
                for t in range(16):
                    w = w16[t]
                    e = g * 16 + t
                    for j in range(D // 16):
                        sl = pl.ds(j * 16, 16)
                        rows_v[e, sl] = rows_v[e, sl] * w
                return c

            lax.fori_loop(0, CHUNK // 16, scale_body, 0)

            pltpu.sync_copy(rows_v, acc.at[row_v], add=True)
            return carry

        lax.fori_loop(0, N_CHUNKS, chunk_body, 0)

        plsc.subcore_barrier()
        pltpu.sync_copy(acc.at[pl.ds(rbase, ROWS_PER_TILE)],
                        out_hbm.at[cid, pl.ds(rbase, ROWS_PER_TILE)])

    return k(x, row, col, ew, zeros)


def _tc_finish(partials, W):
    ROWS_BLK = 2000

    def body(p_ref, w_ref, o_ref):
        o_ref[...] = jnp.dot(p_ref[0] + p_ref[1], w_ref[...],
                             preferred_element_type=jnp.float32)

    return pl.pallas_call(
        body,
        grid=(N_NODES // ROWS_BLK,),
        in_specs=[
            pl.BlockSpec((NC, ROWS_BLK, D), lambda i: (0, i, 0)),
            pl.BlockSpec((D, D), lambda i: (0, 0)),
        ],
        out_specs=pl.BlockSpec((ROWS_BLK, D), lambda i: (i, 0)),
        out_shape=jax.ShapeDtypeStruct((N_NODES, D), jnp.float32),
    )(partials, W)


@jax.jit
def kernel(x, edge_index, edge_weight, W):
    row = edge_index[0].astype(jnp.int32)
    col = edge_index[1].astype(jnp.int32)
    ew = edge_weight.astype(jnp.float32)
    zeros = jnp.zeros((N_PAD, D), jnp.float32)
    partials = _sc_scatter(x, row, col, ew, zeros)
    return _tc_finish(partials, W)


# speedup vs baseline: 11.1280x; 2.6748x over previous
import functools

import jax
import jax.numpy as jnp
from jax import lax
from jax.experimental import pallas as pl
from jax.experimental.pallas import tpu as pltpu
from jax.experimental.pallas import tpu_sc as plsc

N_NODES = 10000
N_EDGES = 320000
D = 128

NC = 2
NS = 16
NW = NC * NS
E_PER_TILE = N_EDGES // NW
CHUNK = 80
N_CHUNKS = E_PER_TILE // CHUNK
NBUF = 3
N_RING = (N_CHUNKS // NBUF) * NBUF
N_PAD = 10112
ROWS_PER_TILE = N_PAD // NS


def _sc_scatter(x, row3, col3, ew3, zeros):
    mesh = plsc.VectorSubcoreMesh(core_axis_name="c", subcore_axis_name="s")

    @functools.partial(
        pl.kernel,
        mesh=mesh,
        out_type=jax.ShapeDtypeStruct((NC, N_PAD, D), jnp.float32),
        scratch_types=[
            pltpu.VMEM_SHARED((N_PAD, D), jnp.float32),
            pltpu.VMEM((N_CHUNKS, CHUNK), jnp.int32),
            [pltpu.VMEM((CHUNK,), jnp.int32) for _ in range(NBUF)],
            [pltpu.VMEM((CHUNK,), jnp.float32) for _ in range(NBUF)],
            [pltpu.VMEM((CHUNK, D), jnp.float32) for _ in range(NBUF)],
            [pltpu.SemaphoreType.DMA for _ in range(NBUF)],
            [pltpu.SemaphoreType.DMA for _ in range(NBUF)],
            [pltpu.SemaphoreType.DMA for _ in range(NBUF)],
        ],
    )
    def k(x_hbm, row_hbm, col_hbm, ew_hbm, z_hbm, out_hbm,
          acc, col_v, row_b, ew_b, bufs, sg, sr, se):
        cid = lax.axis_index("c")
        sid = lax.axis_index("s")
        wid = cid * NS + sid

        pltpu.sync_copy(col_hbm.at[wid], col_v)

        rbase = sid * ROWS_PER_TILE
        pltpu.sync_copy(z_hbm.at[pl.ds(rbase, ROWS_PER_TILE)],
                        acc.at[pl.ds(rbase, ROWS_PER_TILE)])
        plsc.subcore_barrier()

        ebase = wid * E_PER_TILE

        def issue(i, b):
            off = pl.multiple_of(ebase + i * CHUNK, 8)
            pltpu.async_copy(row_hbm.at[pl.ds(off, CHUNK)], row_b[b], sr[b])
            pltpu.async_copy(ew_hbm.at[pl.ds(off, CHUNK)], ew_b[b], se[b])
            pltpu.async_copy(x_hbm.at[col_v.at[i]], bufs[b], sg[b])

        def scale(i, b):
            def scale_body(g, c):
                w16 = ew_b[b][pl.ds(g * 16, 16)]
                for t in range(16):
                    w = w16[t]
                    for j in range(D // 16):
                        sl = pl.ds(j * 16, 16)
                        bufs[b][g * 16 + t, sl] = bufs[b][g * 16 + t, sl] * w
                return c

            lax.fori_loop(0, CHUNK // 16, scale_body, 0)

        def slot(i, b, refill):
            off = pl.multiple_of(ebase + i * CHUNK, 8)
            pltpu.make_async_copy(ew_hbm.at[pl.ds(off, CHUNK)], ew_b[b],
                                  se[b]).wait()
            pltpu.make_async_copy(x_hbm.at[col_v.at[i]], bufs[b],
                                  sg[b]).wait()
            scale(i, b)
            pltpu.make_async_copy(row_hbm.at[pl.ds(off, CHUNK)], row_b[b],
                                  sr[b]).wait()
            pltpu.sync_copy(bufs[b], acc.at[row_b[b]], add=True)
            if refill:
                @pl.when(i + NBUF < N_CHUNKS)
                def _():
                    issue(i + NBUF, b)

        for b in range(NBUF):
            issue(b, b)

        def outer(g, carry):
            for b in range(NBUF):
                slot(g * NBUF + b, b, refill=True)
            return carry

        lax.fori_loop(0, N_RING // NBUF, outer, 0)
        for i in range(N_RING, N_CHUNKS):
            slot(i, i % NBUF, refill=False)

        plsc.subcore_barrier()
        pltpu.sync_copy(acc.at[pl.ds(rbase, ROWS_PER_TILE)],
                        out_hbm.at[cid, pl.ds(rbase, ROWS_PER_TILE)])

    return k(x, row3, col3, ew3, zeros)


def _tc_finish(partials, W):
    ROWS_BLK = 2000

    def body(p_ref, w_ref, o_ref):
        o_ref[...] = jnp.dot(p_ref[0] + p_ref[1], w_ref[...],
                             preferred_element_type=jnp.float32)

    return pl.pallas_call(
        body,
        grid=(N_NODES // ROWS_BLK,),
        in_specs=[
            pl.BlockSpec((NC, ROWS_BLK, D), lambda i: (0, i, 0)),
            pl.BlockSpec((D, D), lambda i: (0, 0)),
        ],
        out_specs=pl.BlockSpec((ROWS_BLK, D), lambda i: (i, 0)),
        out_shape=jax.ShapeDtypeStruct((N_NODES, D), jnp.float32),
    )(partials, W)


@jax.jit
def kernel(x, edge_index, edge_weight, W):
    row = edge_index[0].astype(jnp.int32)
    col = edge_index[1].astype(jnp.int32).reshape(NW, N_CHUNKS, CHUNK)
    ew = edge_weight.astype(jnp.float32)
    zeros = jnp.zeros((N_PAD, D), jnp.float32)
    partials = _sc_scatter(x, row, col, ew, zeros)
    return _tc_finish(partials, W)


# speedup vs baseline: 11.1667x; 1.0035x over previous
import functools

import jax
import jax.numpy as jnp
from jax import lax
from jax.experimental import pallas as pl
from jax.experimental.pallas import tpu as pltpu
from jax.experimental.pallas import tpu_sc as plsc

N_NODES = 10000
N_EDGES = 320000
D = 128

NC = 2
NS = 16
NW = NC * NS
E_PER_TILE = N_EDGES // NW
CHUNK = 80
N_CHUNKS = E_PER_TILE // CHUNK
NBUF = 3
N_RING = (N_CHUNKS // NBUF) * NBUF
N_PAD = 10112
ROWS_PER_TILE = N_PAD // NS


def _sc_scatter(x, row3, col3, ew3, zeros):
    mesh = plsc.VectorSubcoreMesh(core_axis_name="c", subcore_axis_name="s")

    @functools.partial(
        pl.kernel,
        mesh=mesh,
        out_type=jax.ShapeDtypeStruct((NC, N_PAD, D), jnp.float32),
        scratch_types=[
            pltpu.VMEM_SHARED((N_PAD, D), jnp.float32),
            pltpu.VMEM((N_CHUNKS, CHUNK), jnp.int32),
            [pltpu.VMEM((CHUNK,), jnp.int32) for _ in range(NBUF)],
            [pltpu.VMEM((CHUNK,), jnp.float32) for _ in range(NBUF)],
            [pltpu.VMEM((CHUNK, D), jnp.float32) for _ in range(NBUF)],
            [pltpu.SemaphoreType.DMA for _ in range(NBUF)],
            [pltpu.SemaphoreType.DMA for _ in range(NBUF)],
            [pltpu.SemaphoreType.DMA for _ in range(NBUF)],
            [pltpu.SemaphoreType.DMA for _ in range(NBUF)],
        ],
    )
    def k(x_hbm, row_hbm, col_hbm, ew_hbm, z_hbm, out_hbm,
          acc, col_v, row_b, ew_b, bufs, sg, sr, se, ss):
        cid = lax.axis_index("c")
        sid = lax.axis_index("s")
        wid = cid * NS + sid

        pltpu.sync_copy(col_hbm.at[wid], col_v)

        rbase = sid * ROWS_PER_TILE
        pltpu.sync_copy(z_hbm.at[pl.ds(rbase, ROWS_PER_TILE)],
                        acc.at[pl.ds(rbase, ROWS_PER_TILE)])
        plsc.subcore_barrier()

        ebase = wid * E_PER_TILE

        def issue(i, b):
            off = pl.multiple_of(ebase + i * CHUNK, 8)
            pltpu.async_copy(row_hbm.at[pl.ds(off, CHUNK)], row_b[b], sr[b])
            pltpu.async_copy(ew_hbm.at[pl.ds(off, CHUNK)], ew_b[b], se[b])
            pltpu.async_copy(x_hbm.at[col_v.at[i]], bufs[b], sg[b])

        def scale(i, b):
            def scale_body(g, c):
                w16 = ew_b[b][pl.ds(g * 16, 16)]
                for t in range(16):
                    w = w16[t]
                    for j in range(D // 16):
                        sl = pl.ds(j * 16, 16)
                        bufs[b][g * 16 + t, sl] = bufs[b][g * 16 + t, sl] * w
                return c

            lax.fori_loop(0, CHUNK // 16, scale_body, 0)

        def wait_scatter(b):
            pltpu.make_async_copy(bufs[b], acc.at[row_b[b]], ss[b]).wait()

        def slot(i, b, wait_prev=True, do_issue=True):
            off = pl.multiple_of(ebase + i * CHUNK, 8)
            pltpu.make_async_copy(ew_hbm.at[pl.ds(off, CHUNK)], ew_b[b],
                                  se[b]).wait()
            pltpu.make_async_copy(x_hbm.at[col_v.at[i]], bufs[b],
                                  sg[b]).wait()
            scale(i, b)
            pltpu.make_async_copy(row_hbm.at[pl.ds(off, CHUNK)], row_b[b],
                                  sr[b]).wait()
            pltpu.async_copy(bufs[b], acc.at[row_b[b]], ss[b], add=True)
            if do_issue:
                b2 = (b + 2) % NBUF
                if wait_prev:
                    wait_scatter(b2)
                issue(i + 2, b2)

        for b in range(2):
            issue(b, b)
        slot(0, 0, wait_prev=False)
        slot(1, 1)

        def outer(g, carry):
            for d in range(NBUF):
                slot(g * NBUF + 2 + d, (2 + d) % NBUF)
            return carry

        lax.fori_loop(0, 40, outer, 0)
        slot(122, 122 % NBUF)
        slot(123, 123 % NBUF, do_issue=False)
        slot(124, 124 % NBUF, do_issue=False)
        for b in range(NBUF):
            wait_scatter(b)

        plsc.subcore_barrier()
        pltpu.sync_copy(acc.at[pl.ds(rbase, ROWS_PER_TILE)],
                        out_hbm.at[cid, pl.ds(rbase, ROWS_PER_TILE)])

    return k(x, row3, col3, ew3, zeros)


def _tc_finish(partials, W):
    ROWS_BLK = 2000

    def body(p_ref, w_ref, o_ref):
        o_ref[...] = jnp.dot(p_ref[0] + p_ref[1], w_ref[...],
                             preferred_element_type=jnp.float32)

    return pl.pallas_call(
        body,
        grid=(N_NODES // ROWS_BLK,),
        in_specs=[
            pl.BlockSpec((NC, ROWS_BLK, D), lambda i: (0, i, 0)),
            pl.BlockSpec((D, D), lambda i: (0, 0)),
        ],
        out_specs=pl.BlockSpec((ROWS_BLK, D), lambda i: (i, 0)),
        out_shape=jax.ShapeDtypeStruct((N_NODES, D), jnp.float32),
    )(partials, W)


@jax.jit
def kernel(x, edge_index, edge_weight, W):
    row = edge_index[0].astype(jnp.int32)
    col = edge_index[1].astype(jnp.int32).reshape(NW, N_CHUNKS, CHUNK)
    ew = edge_weight.astype(jnp.float32)
    zeros = jnp.zeros((N_PAD, D), jnp.float32)
    partials = _sc_scatter(x, row, col, ew, zeros)
    return _tc_finish(partials, W)


# speedup vs baseline: 12.0185x; 1.0763x over previous
import functools

import jax
import jax.numpy as jnp
from jax import lax
from jax.experimental import pallas as pl
from jax.experimental.pallas import tpu as pltpu
from jax.experimental.pallas import tpu_sc as plsc

N_NODES = 10000
N_EDGES = 320000
D = 128

NC = 2
NS = 16
NW = NC * NS
E_PER_TILE = N_EDGES // NW
CHUNK = 80
N_CHUNKS = E_PER_TILE // CHUNK
NB = 4
N_PAD = 10112
ROWS_PER_TILE = N_PAD // NS


def _sc_scatter(x, ei, ew, zeros):
    mesh = plsc.VectorSubcoreMesh(core_axis_name="c", subcore_axis_name="s")

    @functools.partial(
        pl.kernel,
        mesh=mesh,
        out_type=jax.ShapeDtypeStruct((NC, N_PAD, D), jnp.float32),
        scratch_types=[
            pltpu.VMEM_SHARED((N_PAD, D), jnp.float32),
            [pltpu.VMEM((CHUNK,), jnp.int32) for _ in range(NB)],
            [pltpu.VMEM((CHUNK,), jnp.int32) for _ in range(NB)],
            [pltpu.VMEM((CHUNK,), jnp.float32) for _ in range(NB)],
            [pltpu.VMEM((CHUNK, D), jnp.float32) for _ in range(NB)],
            [pltpu.SemaphoreType.DMA for _ in range(NB)],
            [pltpu.SemaphoreType.DMA for _ in range(NB)],
            [pltpu.SemaphoreType.DMA for _ in range(NB)],
            [pltpu.SemaphoreType.DMA for _ in range(NB)],
        ],
    )
    def k(x_hbm, ei_hbm, ew_hbm, z_hbm, out_hbm,
          acc, colb, rowb, ewb, gbufs, scol, sr, se, sg):
        cid = lax.axis_index("c")
        sid = lax.axis_index("s")
        wid = cid * NS + sid

        rbase = sid * ROWS_PER_TILE
        pltpu.sync_copy(z_hbm.at[pl.ds(rbase, ROWS_PER_TILE)],
                        acc.at[pl.ds(rbase, ROWS_PER_TILE)])
        plsc.subcore_barrier()

        ebase = wid * E_PER_TILE

        def row_off(i):
            return pl.multiple_of(ebase + i * CHUNK, 8)

        def col_off(i):
            return pl.multiple_of(N_EDGES + ebase + i * CHUNK, 8)

        def issue_col(i, p):
            pltpu.async_copy(ei_hbm.at[pl.ds(col_off(i), CHUNK)],
                             colb[p], scol[p])

        def issue_row_ew(i, p):
            pltpu.async_copy(ei_hbm.at[pl.ds(row_off(i), CHUNK)],
                             rowb[p], sr[p])
            pltpu.async_copy(ew_hbm.at[pl.ds(row_off(i), CHUNK)],
                             ewb[p], se[p])

        def issue_gather(i, p):
            pltpu.make_async_copy(ei_hbm.at[pl.ds(col_off(i), CHUNK)],
                                  colb[p], scol[p]).wait()
            pltpu.async_copy(x_hbm.at[colb[p]], gbufs[p], sg[p])

        def scale(i, p):
            def g_body(g, c):
                w16 = ewb[p][pl.ds(g * 16, 16)]
                for t in range(16):
                    w = w16[t]
                    for j in range(D // 16):
                        sl = pl.ds(j * 16, 16)
                        gbufs[p][g * 16 + t, sl] = gbufs[p][g * 16 + t, sl] * w
                return c

            lax.fori_loop(0, CHUNK // 16, g_body, 0)

        def slot(i, p, do_col=True, do_gather=True, do_row_ew=True):
            if do_col:
                issue_col(i + 5, (p + 1) % NB)
            if do_gather:
                issue_gather(i + 3, (p + 3) % NB)
            pltpu.make_async_copy(ew_hbm.at[pl.ds(row_off(i), CHUNK)],
                                  ewb[p], se[p]).wait()
            pltpu.make_async_copy(x_hbm.at[colb[p]], gbufs[p], sg[p]).wait()
            scale(i, p)
            pltpu.make_async_copy(ei_hbm.at[pl.ds(row_off(i), CHUNK)],
                                  rowb[p], sr[p]).wait()
            pltpu.sync_copy(gbufs[p], acc.at[rowb[p]], add=True)
            if do_row_ew:
                issue_row_ew(i + 4, p)

        for i2 in range(NB):
            issue_col(i2, i2)
        for i2 in range(3):
            issue_gather(i2, i2)
        issue_col(4, 0)
        for i2 in range(NB):
            issue_row_ew(i2, i2)

        def outer(g, carry):
            for d in range(NB):
                slot(g * NB + d, d)
            return carry

        lax.fori_loop(0, 30, outer, 0)
        slot(120, 0, do_col=False)
        slot(121, 1, do_col=False, do_row_ew=False)
        slot(122, 2, do_col=False, do_gather=False, do_row_ew=False)
        slot(123, 3, do_col=False, do_gather=False, do_row_ew=False)
        slot(124, 0, do_col=False, do_gather=False, do_row_ew=False)

        plsc.subcore_barrier()
        pltpu.sync_copy(acc.at[pl.ds(rbase, ROWS_PER_TILE)],
                        out_hbm.at[cid, pl.ds(rbase, ROWS_PER_TILE)])

    return k(x, ei, ew, zeros)


def _tc_finish(partials, W):
    ROWS_BLK = 2000

    def body(p_ref, w_ref, o_ref):
        o_ref[...] = jnp.dot(p_ref[0] + p_ref[1], w_ref[...],
                             preferred_element_type=jnp.float32)

    return pl.pallas_call(
        body,
        grid=(N_NODES // ROWS_BLK,),
        in_specs=[
            pl.BlockSpec((NC, ROWS_BLK, D), lambda i: (0, i, 0)),
            pl.BlockSpec((D, D), lambda i: (0, 0)),
        ],
        out_specs=pl.BlockSpec((ROWS_BLK, D), lambda i: (i, 0)),
        out_shape=jax.ShapeDtypeStruct((N_NODES, D), jnp.float32),
    )(partials, W)


@jax.jit
def kernel(x, edge_index, edge_weight, W):
    ei = edge_index.astype(jnp.int32).reshape(2 * N_EDGES)
    ew = edge_weight.astype(jnp.float32)
    zeros = jnp.zeros((N_PAD, D), jnp.float32)
    partials = _sc_scatter(x, ei, ew, zeros)
    return _tc_finish(partials, W)
